# baseline (speedup 1.0000x reference)
import functools

import jax
import jax.numpy as jnp
from jax import lax
from jax.experimental import pallas as pl

_NC = 81
_N = 20000
_K = 64
_CHUNK = 160
_GRID = _N // _CHUNK
_PAD = 20096
_ROWS = _PAD // 128
_MIN32 = -2147483648


def _rowstats_kernel(s_ref, lab_ref, pos_ref, neg_ref, cfg_ref, cbg_ref):
    s = s_ref[...]
    lab = lab_ref[...]
    col = lax.broadcasted_iota(jnp.int32, s.shape, 1)
    neginf = jnp.float32(-jnp.inf)

    m_ex = jnp.max(jnp.where(col != _NC - 1, s, neginf), axis=1, keepdims=True)
    metric = -m_ex
    fg = lab != _NC
    pos_ref[...] = jnp.where(fg, metric, neginf)
    neg_ref[...] = jnp.where(fg, neginf, metric)

    m_all = jnp.max(s, axis=1, keepdims=True)
    e = jnp.exp(s - m_all)
    ssum = jnp.sum(e, axis=1, keepdims=True)
    e_lab = jnp.sum(jnp.where(col == lab, e, 0.0), axis=1, keepdims=True)
    gt = e_lab / ssum
    t = gt * (1.0 - gt)
    denomlog = m_all + jnp.log(ssum - e_lab)
    s79 = s[:, _NC - 2:_NC - 1]
    s80 = s[:, _NC - 1:_NC]
    s81 = s[:, _NC:_NC + 1]
    c_fg = jnp.where(lab <= _NC - 2, s80, s79)
    c_bg = jnp.where(lab <= _NC - 1, s81, s80)
    cfg_ref[...] = t * (c_fg - denomlog)
    cbg_ref[...] = t * (c_bg - denomlog)


def _sortable(x):
    b = lax.bitcast_convert_type(x, jnp.int32)
    return jnp.where(b < 0, b ^ jnp.int32(0x7FFFFFFF), b)


def _select_kernel(pos_ref, neg_ref, cfg_ref, cbg_ref, out_ref):
    kp = _sortable(pos_ref[...])
    kn = _sortable(neg_ref[...])
    k = jnp.int32(_K)

    def bit_body(t, carry):
        vp, vn = carry
        bit = lax.shift_left(jnp.int32(1), jnp.int32(31) - t)
        candp = vp | bit
        candn = vn | bit
        thrp = candp ^ jnp.int32(_MIN32)
        thrn = candn ^ jnp.int32(_MIN32)
        cntp = jnp.sum((kp >= thrp).astype(jnp.int32))
        cntn = jnp.sum((kn >= thrn).astype(jnp.int32))
        vp = jnp.where(cntp >= k, candp, vp)
        vn = jnp.where(cntn >= k, candn, vn)
        return vp, vn

    vp_u, vn_u = lax.fori_loop(0, 32, bit_body, (jnp.int32(0), jnp.int32(0)))
    vp = vp_u ^ jnp.int32(_MIN32)
    vn = vn_u ^ jnp.int32(_MIN32)

    gtp = kp > vp
    gtn = kn > vn
    tiep = kp == vp
    tien = kn == vn
    fillp = k - jnp.sum(gtp.astype(jnp.int32))
    filln = k - jnp.sum(gtn.astype(jnp.int32))

    idx = (lax.broadcasted_iota(jnp.int32, (_ROWS, 128), 0) * 128
           + lax.broadcasted_iota(jnp.int32, (_ROWS, 128), 1))

    def idx_body(_, carry):
        lop, hip, lon, hin = carry
        midp = (lop + hip) // 2
        midn = (lon + hin) // 2
        cp = jnp.sum((tiep & (idx < midp)).astype(jnp.int32))
        cn = jnp.sum((tien & (idx < midn)).astype(jnp.int32))
        okp = cp >= fillp
        okn = cn >= filln
        return (jnp.where(okp, lop, midp), jnp.where(okp, midp, hip),
                jnp.where(okn, lon, midn), jnp.where(okn, midn, hin))

    z = jnp.int32(0)
    hi0 = jnp.int32(32768)
    lop, hip, lon, hin = lax.fori_loop(0, 15, idx_body, (z, hi0, z, hi0))

    selp = gtp | (tiep & (idx < hip))
    seln = gtn | (tien & (idx < hin))
    sump = jnp.sum(jnp.where(selp, cfg_ref[...], 0.0))
    sumn = jnp.sum(jnp.where(seln, cbg_ref[...], 0.0))
    loss = -(sump + sumn) / jnp.float32(2 * _K)
    out_ref[...] = jnp.broadcast_to(loss, (1, 1))


@jax.jit
def kernel(scores, labels, un_id, weight, bias):
    lab2 = labels.reshape(_N, 1)
    pos, neg, cfg, cbg = pl.pallas_call(
        _rowstats_kernel,
        grid=(_GRID,),
        in_specs=[
            pl.BlockSpec((_CHUNK, _NC + 1), lambda i: (i, 0)),
            pl.BlockSpec((_CHUNK, 1), lambda i: (i, 0)),
        ],
        out_specs=[pl.BlockSpec((_CHUNK, 1), lambda i: (i, 0))] * 4,
        out_shape=[jax.ShapeDtypeStruct((_N, 1), jnp.float32)] * 4,
    )(scores, lab2)

    def prep(x, padval):
        flat = x.reshape(_N)
        padded = jnp.concatenate(
            [flat, jnp.full((_PAD - _N,), padval, jnp.float32)])
        return padded.reshape(_ROWS, 128)

    neginf = jnp.float32(-jnp.inf)
    loss = pl.pallas_call(
        _select_kernel,
        out_shape=jax.ShapeDtypeStruct((1, 1), jnp.float32),
    )(prep(pos, neginf), prep(neg, neginf), prep(cfg, 0.0), prep(cbg, 0.0))
    return loss.reshape(())


# speedup vs baseline: 6.3389x; 6.3389x over previous
import jax
import jax.numpy as jnp
from jax import lax
from jax.experimental import pallas as pl

_NC = 81
_N = 20000
_K = 64
_NP = 20480
_CHUNK = 2048
_GRID = _NP // _CHUNK
_ROWS = _NP // 128
_MIN32 = -2147483648


def _rowstats_kernel(st_ref, lab_ref, pos_ref, neg_ref, cfg_ref, cbg_ref):
    s = st_ref[...]
    lab = lab_ref[...]
    gcol = (pl.program_id(0) * _CHUNK
            + lax.broadcasted_iota(jnp.int32, (1, _CHUNK), 1))
    valid = gcol < _N

    m80 = jnp.max(s[:_NC - 1, :], axis=0, keepdims=True)
    m_ex = jnp.maximum(m80, s[_NC:_NC + 1, :])
    metric = -m_ex
    b = lax.bitcast_convert_type(metric, jnp.int32)
    key = jnp.where(b < 0, b ^ jnp.int32(0x7FFFFFFF), b)
    fg = (lab != _NC) & valid
    bg = (lab == _NC) & valid
    masked = jnp.int32(_MIN32)
    pos_ref[...] = jnp.where(fg, key, masked)
    neg_ref[...] = jnp.where(bg, key, masked)

    m_all = jnp.maximum(m_ex, s[_NC - 1:_NC, :])
    e = jnp.exp(s - m_all)
    ssum = jnp.sum(e, axis=0, keepdims=True)
    row = lax.broadcasted_iota(jnp.int32, s.shape, 0)
    e_lab = jnp.sum(jnp.where(row == lab, e, 0.0), axis=0, keepdims=True)
    gt = e_lab / ssum
    t = gt * (1.0 - gt)
    denomlog = m_all + jnp.log(ssum - e_lab)
    s79 = s[_NC - 2:_NC - 1, :]
    s80 = s[_NC - 1:_NC, :]
    s81 = s[_NC:_NC + 1, :]
    c_fg = jnp.where(lab <= _NC - 2, s80, s79)
    c_bg = jnp.where(lab <= _NC - 1, s81, s80)
    cfg_ref[...] = t * (c_fg - denomlog)
    cbg_ref[...] = t * (c_bg - denomlog)


def _count_ge(keys, thr_u):
    return jnp.sum((keys >= (thr_u ^ jnp.int32(_MIN32))).astype(jnp.int32))


def _radix_kth(keys, k):
    def step(t, v):
        shift = 28 - 4 * t
        nb = jnp.int32(0)
        for j in range(1, 16):
            cand = v | lax.shift_left(jnp.int32(j), shift)
            nb = nb + (_count_ge(keys, cand) >= k).astype(jnp.int32)
        return v | lax.shift_left(nb, shift)
    return lax.fori_loop(0, 8, step, jnp.int32(0))


def _tie_T(tie, idx, fill):
    def step(t, v):
        shift = 12 - 4 * t
        nb = jnp.int32(0)
        for j in range(1, 16):
            cand = v | lax.shift_left(jnp.int32(j), shift)
            cnt = jnp.sum((tie & (idx < cand)).astype(jnp.int32))
            nb = nb + (cnt < fill).astype(jnp.int32)
        return v | lax.shift_left(nb, shift)
    return lax.fori_loop(0, 4, step, jnp.int32(0)) + 1


def _select_kernel(pos_ref, neg_ref, cfg_ref, cbg_ref, out_ref):
    kp = pos_ref[...]
    kn = neg_ref[...]
    k = jnp.int32(_K)

    vp = _radix_kth(kp, k) ^ jnp.int32(_MIN32)
    vn = _radix_kth(kn, k) ^ jnp.int32(_MIN32)

    gtp = kp > vp
    gtn = kn > vn
    tiep = kp == vp
    tien = kn == vn
    fillp = k - jnp.sum(gtp.astype(jnp.int32))
    filln = k - jnp.sum(gtn.astype(jnp.int32))

    idx = (lax.broadcasted_iota(jnp.int32, (_ROWS, 128), 0) * 128
           + lax.broadcasted_iota(jnp.int32, (_ROWS, 128), 1))

    tp = _tie_T(tiep, idx, fillp)
    tn = _tie_T(tien, idx, filln)

    selp = gtp | (tiep & (idx < tp))
    seln = gtn | (tien & (idx < tn))
    sump = jnp.sum(jnp.where(selp, cfg_ref[...], 0.0))
    sumn = jnp.sum(jnp.where(seln, cbg_ref[...], 0.0))
    loss = -(sump + sumn) / jnp.float32(2 * _K)
    out_ref[...] = jnp.broadcast_to(loss, (1, 1))


@jax.jit
def kernel(scores, labels, un_id, weight, bias):
    st = jnp.pad(scores.T, ((0, 0), (0, _NP - _N)))
    lab2 = jnp.pad(labels.reshape(1, _N), ((0, 0), (0, _NP - _N)))
    pos, neg, cfg, cbg = pl.pallas_call(
        _rowstats_kernel,
        grid=(_GRID,),
        in_specs=[
            pl.BlockSpec((_NC + 1, _CHUNK), lambda i: (0, i)),
            pl.BlockSpec((1, _CHUNK), lambda i: (0, i)),
        ],
        out_specs=[pl.BlockSpec((1, _CHUNK), lambda i: (0, i))] * 4,
        out_shape=[jax.ShapeDtypeStruct((1, _NP), jnp.int32)] * 2
        + [jax.ShapeDtypeStruct((1, _NP), jnp.float32)] * 2,
    )(st, lab2)

    loss = pl.pallas_call(
        _select_kernel,
        out_shape=jax.ShapeDtypeStruct((1, 1), jnp.float32),
    )(pos.reshape(_ROWS, 128), neg.reshape(_ROWS, 128),
      cfg.reshape(_ROWS, 128), cbg.reshape(_ROWS, 128))
    return loss.reshape(())
